# baseline (speedup 1.0000x reference)
import functools

import jax
import jax.numpy as jnp
from jax import lax
from jax.experimental import pallas as pl
from jax.experimental.pallas import tpu as pltpu
from jax.experimental.pallas import tpu_sc as plsc

NC = 2
NS = 16
NW = NC * NS
LANES = 16

VOCAB_DIM = 300
PIECE = 128
BATCH = 16384
PER_W = BATCH // NW
CHUNK = 64
NCHUNK = PER_W // CHUNK
NBUF = 2


def _sc_body(x0_hbm, x1_hbm, tbl_hbm, out_hbm,
             idx0_v, idx1_v, a0_v, b0_v, a1_v, b1_v,
             stage_v, s0, s1, sg):
    wid = lax.axis_index("s") * NC + lax.axis_index("c")
    base = wid * PER_W
    ca = pltpu.async_copy(x0_hbm.at[pl.ds(base, PER_W)], idx0_v, s0)
    cb = pltpu.async_copy(x1_hbm.at[pl.ds(base, PER_W)], idx1_v, s1)
    ca.wait()
    cb.wait()

    bufs = [(a0_v, b0_v), (a1_v, b1_v)]
    sems = [s0, s1]

    def issue(g):
        slot = g % NBUF
        i0 = idx0_v.at[pl.ds(g * CHUNK, CHUNK)]
        i1 = idx1_v.at[pl.ds(g * CHUNK, CHUNK)]
        av, bv = bufs[slot]
        sem = sems[slot]
        return (
            pltpu.async_copy(tbl_hbm.at[i0], av, sem),
            pltpu.async_copy(tbl_hbm.at[i1], bv, sem),
        )

    acc = jnp.zeros((LANES,), jnp.float32)
    pending = {}
    for g in range(min(NBUF, NCHUNK)):
        pending[g] = issue(g)
    for g in range(NCHUNK):
        slot = g % NBUF
        for h in pending.pop(g):
            h.wait()
        av, bv = bufs[slot]

        def row_step(r, acc, av=av, bv=bv):
            for j in range(PIECE // LANES):
                a = av[r, pl.ds(j * LANES, LANES)]
                b = bv[r, pl.ds(j * LANES, LANES)]
                acc = acc + a * b
            return acc

        acc = lax.fori_loop(0, CHUNK, row_step, acc)
        if g + NBUF < NCHUNK:
            pending[g + NBUF] = issue(g + NBUF)

    for i in range(8):
        for j in range(128 // LANES):
            stage_v[i, pl.ds(j * LANES, LANES)] = jnp.zeros(
                (LANES,), jnp.float32)
    stage_v[0, pl.ds(0, LANES)] = acc
    pltpu.async_copy(stage_v, out_hbm.at[wid], sg).wait()


@jax.jit
def _sc_gather_dot(x0, x1, piece):
    mesh = plsc.VectorSubcoreMesh(core_axis_name="c", subcore_axis_name="s")
    return pl.kernel(
        _sc_body,
        out_type=jax.ShapeDtypeStruct((NW, 8, 128), jnp.float32),
        mesh=mesh,
        scratch_types=[
            pltpu.VMEM((PER_W,), jnp.int32),
            pltpu.VMEM((PER_W,), jnp.int32),
            pltpu.VMEM((CHUNK, PIECE), jnp.float32),
            pltpu.VMEM((CHUNK, PIECE), jnp.float32),
            pltpu.VMEM((CHUNK, PIECE), jnp.float32),
            pltpu.VMEM((CHUNK, PIECE), jnp.float32),
            pltpu.VMEM((8, 128), jnp.float32),
            pltpu.SemaphoreType.DMA,
            pltpu.SemaphoreType.DMA,
            pltpu.SemaphoreType.DMA,
        ],
    )(x0, x1, piece)


def _finish_body(p0_ref, p1_ref, p2_ref, o_ref):
    s = jnp.sum(p0_ref[...]) + jnp.sum(p1_ref[...]) + jnp.sum(p2_ref[...])
    o_ref[0, 0] = s * s


@jax.jit
def _finish(p0, p1, p2):
    out = pl.pallas_call(
        _finish_body,
        out_shape=jax.ShapeDtypeStruct((1, 1), jnp.float32),
        out_specs=pl.BlockSpec(memory_space=pltpu.SMEM),
    )(p0, p1, p2)
    return out[0, 0]


def kernel(x, table):
    x0 = x[:, 0]
    x1 = x[:, 1]
    tt = table.T
    pieces = [
        tt[0:PIECE].T,
        tt[PIECE:2 * PIECE].T,
        jnp.pad(tt[2 * PIECE:VOCAB_DIM].T,
                ((0, 0), (0, 3 * PIECE - VOCAB_DIM))),
    ]
    partials = [_sc_gather_dot(x0, x1, p) for p in pieces]
    return _finish(*partials)


# speedup vs baseline: 2.1362x; 2.1362x over previous
import functools

import jax
import jax.numpy as jnp
from jax import lax
from jax.experimental import pallas as pl
from jax.experimental.pallas import tpu as pltpu
from jax.experimental.pallas import tpu_sc as plsc

NC = 2
NS = 16
NW = NC * NS
LANES = 16

VOCAB_DIM = 300
PIECE = 128
BATCH = 16384
PER_W = BATCH // NW
CHUNK = 64
NCHUNK = PER_W // CHUNK
NBUF = 2


def _sc_body(x0_hbm, x1_hbm, tbl_hbm, out_hbm,
             idx0_v, idx1_v, a0_v, b0_v, a1_v, b1_v,
             stage_v, s0, s1, sg):
    wid = lax.axis_index("s") * NC + lax.axis_index("c")
    base = wid * PER_W
    ca = pltpu.async_copy(x0_hbm.at[pl.ds(base, PER_W)], idx0_v, s0)
    cb = pltpu.async_copy(x1_hbm.at[pl.ds(base, PER_W)], idx1_v, s1)
    ca.wait()
    cb.wait()

    bufs = [(a0_v, b0_v), (a1_v, b1_v)]
    sems = [s0, s1]

    def issue(g):
        slot = g % NBUF
        i0 = idx0_v.at[pl.ds(g * CHUNK, CHUNK)]
        i1 = idx1_v.at[pl.ds(g * CHUNK, CHUNK)]
        av, bv = bufs[slot]
        sem = sems[slot]
        return (
            pltpu.async_copy(tbl_hbm.at[i0], av, sem),
            pltpu.async_copy(tbl_hbm.at[i1], bv, sem),
        )

    acc = jnp.zeros((LANES,), jnp.float32)
    pending = {}
    for g in range(min(NBUF, NCHUNK)):
        pending[g] = issue(g)
    for g in range(NCHUNK):
        slot = g % NBUF
        for h in pending.pop(g):
            h.wait()
        av, bv = bufs[slot]

        def row_step(r, acc, av=av, bv=bv):
            for j in range(PIECE // LANES):
                a = av[r, pl.ds(j * LANES, LANES)]
                b = bv[r, pl.ds(j * LANES, LANES)]
                acc = acc + a * b
            return acc

        acc = lax.fori_loop(0, CHUNK, row_step, acc)
        if g + NBUF < NCHUNK:
            pending[g + NBUF] = issue(g + NBUF)

    for i in range(8):
        for j in range(128 // LANES):
            stage_v[i, pl.ds(j * LANES, LANES)] = jnp.zeros(
                (LANES,), jnp.float32)
    stage_v[0, pl.ds(0, LANES)] = acc
    pltpu.async_copy(stage_v, out_hbm.at[wid], sg).wait()


@jax.jit
def _sc_gather_dot(x0, x1, piece):
    mesh = plsc.VectorSubcoreMesh(core_axis_name="c", subcore_axis_name="s")
    return pl.kernel(
        _sc_body,
        out_type=jax.ShapeDtypeStruct((NW, 8, 128), jnp.float32),
        mesh=mesh,
        scratch_types=[
            pltpu.VMEM((PER_W,), jnp.int32),
            pltpu.VMEM((PER_W,), jnp.int32),
            pltpu.VMEM((CHUNK, PIECE), jnp.float32),
            pltpu.VMEM((CHUNK, PIECE), jnp.float32),
            pltpu.VMEM((CHUNK, PIECE), jnp.float32),
            pltpu.VMEM((CHUNK, PIECE), jnp.float32),
            pltpu.VMEM((8, 128), jnp.float32),
            pltpu.SemaphoreType.DMA,
            pltpu.SemaphoreType.DMA,
            pltpu.SemaphoreType.DMA,
        ],
    )(x0, x1, piece)


def _finish_body(p0_ref, p1_ref, p2_ref, o_ref):
    s = jnp.sum(p0_ref[...]) + jnp.sum(p1_ref[...]) + jnp.sum(p2_ref[...])
    o_ref[0, 0] = s * s


@jax.jit
def _finish(p0, p1, p2):
    out = pl.pallas_call(
        _finish_body,
        out_shape=jax.ShapeDtypeStruct((1, 1), jnp.float32),
        out_specs=pl.BlockSpec(memory_space=pltpu.SMEM),
    )(p0, p1, p2)
    return out[0, 0]


def kernel(x, table):
    x0 = x[:, 0]
    x1 = x[:, 1]
    pieces = [
        table[:, 0:PIECE],
        table[:, PIECE:2 * PIECE],
        jnp.pad(table[:, 2 * PIECE:VOCAB_DIM],
                ((0, 0), (0, 3 * PIECE - VOCAB_DIM))),
    ]
    partials = [_sc_gather_dot(x0, x1, p) for p in pieces]
    return _finish(*partials)


# speedup vs baseline: 2.3381x; 1.0945x over previous
import functools

import jax
import jax.numpy as jnp
from jax import lax
from jax.experimental import pallas as pl
from jax.experimental.pallas import tpu as pltpu
from jax.experimental.pallas import tpu_sc as plsc

NC = 2
NS = 16
NW = NC * NS
LANES = 16

VOCAB = 28436
VOCAB_DIM = 300
DP = 384
NSLICE = 19
VB = 512
NVB = (VOCAB + VB - 1) // VB
BATCH = 16384
PER_W = BATCH // NW
CHUNK = 64
NCHUNK = PER_W // CHUNK
NBUF = 2


def _transpose_body(in_ref, out_ref):
    x = in_ref[...]
    y = jnp.swapaxes(x, 0, 1)
    z = jnp.zeros((VB, DP - VOCAB_DIM), jnp.float32)
    out_ref[...] = jnp.concatenate([y, z], axis=1)


@jax.jit
def _transpose_pad(tt):
    return pl.pallas_call(
        _transpose_body,
        grid=(NVB,),
        in_specs=[pl.BlockSpec((VOCAB_DIM, VB), lambda i: (0, i))],
        out_specs=pl.BlockSpec((VB, DP), lambda i: (i, 0)),
        out_shape=jax.ShapeDtypeStruct((VOCAB, DP), jnp.float32),
    )(tt)


def _sc_body(x0_hbm, x1_hbm, tbl_hbm, out_hbm,
             idx0_v, idx1_v, a0_v, b0_v, a1_v, b1_v,
             stage_v, s0, s1, sg):
    wid = lax.axis_index("s") * NC + lax.axis_index("c")
    base = wid * PER_W
    ca = pltpu.async_copy(x0_hbm.at[pl.ds(base, PER_W)], idx0_v, s0)
    cb = pltpu.async_copy(x1_hbm.at[pl.ds(base, PER_W)], idx1_v, s1)
    ca.wait()
    cb.wait()

    bufs = [(a0_v, b0_v), (a1_v, b1_v)]
    sems = [s0, s1]

    def issue(g):
        slot = g % NBUF
        i0 = idx0_v.at[pl.ds(g * CHUNK, CHUNK)]
        i1 = idx1_v.at[pl.ds(g * CHUNK, CHUNK)]
        av, bv = bufs[slot]
        sem = sems[slot]
        return (
            pltpu.async_copy(tbl_hbm.at[i0], av, sem),
            pltpu.async_copy(tbl_hbm.at[i1], bv, sem),
        )

    acc = jnp.zeros((LANES,), jnp.float32)
    pending = {}
    for g in range(min(NBUF, NCHUNK)):
        pending[g] = issue(g)
    for g in range(NCHUNK):
        slot = g % NBUF
        for h in pending.pop(g):
            h.wait()
        av, bv = bufs[slot]

        def row_step(r, acc, av=av, bv=bv):
            for j in range(NSLICE):
                a = av[r, pl.ds(j * LANES, LANES)]
                b = bv[r, pl.ds(j * LANES, LANES)]
                acc = acc + a * b
            return acc

        acc = lax.fori_loop(0, CHUNK, row_step, acc)
        if g + NBUF < NCHUNK:
            pending[g + NBUF] = issue(g + NBUF)

    for i in range(8):
        for j in range(128 // LANES):
            stage_v[i, pl.ds(j * LANES, LANES)] = jnp.zeros(
                (LANES,), jnp.float32)
    stage_v[0, pl.ds(0, LANES)] = acc
    pltpu.async_copy(stage_v, out_hbm.at[wid], sg).wait()


@jax.jit
def _sc_gather_dot(x0, x1, table_p):
    mesh = plsc.VectorSubcoreMesh(core_axis_name="c", subcore_axis_name="s")
    return pl.kernel(
        _sc_body,
        out_type=jax.ShapeDtypeStruct((NW, 8, 128), jnp.float32),
        mesh=mesh,
        scratch_types=[
            pltpu.VMEM((PER_W,), jnp.int32),
            pltpu.VMEM((PER_W,), jnp.int32),
            pltpu.VMEM((CHUNK, DP), jnp.float32),
            pltpu.VMEM((CHUNK, DP), jnp.float32),
            pltpu.VMEM((CHUNK, DP), jnp.float32),
            pltpu.VMEM((CHUNK, DP), jnp.float32),
            pltpu.VMEM((8, 128), jnp.float32),
            pltpu.SemaphoreType.DMA,
            pltpu.SemaphoreType.DMA,
            pltpu.SemaphoreType.DMA,
        ],
    )(x0, x1, table_p)


def _finish_body(p_ref, o_ref):
    s = jnp.sum(p_ref[...])
    o_ref[0, 0] = s * s


@jax.jit
def _finish(partials):
    out = pl.pallas_call(
        _finish_body,
        out_shape=jax.ShapeDtypeStruct((1, 1), jnp.float32),
        out_specs=pl.BlockSpec(memory_space=pltpu.SMEM),
    )(partials)
    return out[0, 0]


def kernel(x, table):
    x0 = x[:, 0]
    x1 = x[:, 1]
    table_p = _transpose_pad(table.T)
    partials = _sc_gather_dot(x0, x1, table_p)
    return _finish(partials)
